# baseline (speedup 1.0000x reference)
import jax
import jax.numpy as jnp
from jax import lax
from jax.experimental import pallas as pl
from jax.experimental.pallas import tpu as pltpu
from jax.experimental.pallas import tpu_sc as plsc

N = 50000
E = 800000
CELL_INV = 1.0 / 16.0
GRID_W = 8
NUM_GRIDS = 64
EPS = 1e-5

NPAD = 51200
RPT = NPAD // 16
EPAD = 802816
EBLK = 256
NBW = EPAD // 16 // EBLK
NQ = 4
NBQ = NBW // NQ
BA = 2048
NBLK = NPAD // BA
FH = 16


def _make_seg_sum():
    mesh = plsc.VectorSubcoreMesh(core_axis_name="c", subcore_axis_name="s",
                                  num_cores=1)

    def body(tab_hbm, src_hbm, dst_hbm, z_hbm, s_out,
             srcv, dstv, rowsu, semu, sems, acc):
        t = lax.axis_index("s")
        pltpu.sync_copy(z_hbm, acc.at[pl.ds(t * RPT, RPT)])
        plsc.subcore_barrier()

        def chunk(q, carry0):
            pltpu.sync_copy(src_hbm.at[t, pl.ds(q * NBQ, NBQ)], srcv)
            pltpu.sync_copy(dst_hbm.at[t, pl.ds(q * NBQ, NBQ)], dstv)
            for p in range(3):
                pltpu.async_copy(tab_hbm.at[srcv.at[p]], rowsu.at[p], semu)

            def step(j, carry):
                @pl.when(j >= 1)
                def _():
                    pltpu.make_async_copy(
                        rowsu.at[0], acc.at[dstv.at[0]], sems).wait()

                pltpu.make_async_copy(
                    tab_hbm.at[srcv.at[j]], rowsu.at[j % 4], semu).wait()

                pltpu.async_copy(rowsu.at[j % 4], acc.at[dstv.at[j]],
                                 sems, add=True)

                @pl.when(j + 3 < NBQ)
                def _():
                    pltpu.async_copy(
                        tab_hbm.at[srcv.at[j + 3]], rowsu.at[(j + 3) % 4],
                        semu)

                return carry

            lax.fori_loop(0, NBQ, step, 0)
            pltpu.make_async_copy(rowsu.at[0], acc.at[dstv.at[0]],
                                  sems).wait()
            return carry0

        lax.fori_loop(0, NQ, chunk, 0)
        plsc.subcore_barrier()
        pltpu.sync_copy(acc.at[pl.ds(t * RPT, RPT)],
                        s_out.at[pl.ds(t * RPT, RPT)])

    return pl.kernel(
        body,
        out_type=(jax.ShapeDtypeStruct((NPAD, FH), jnp.float32),),
        scratch_types=[
            pltpu.VMEM((NBQ, EBLK), jnp.int32),
            pltpu.VMEM((NBQ, EBLK), jnp.int32),
            pltpu.VMEM((4, EBLK, FH), jnp.float32),
            pltpu.SemaphoreType.DMA,
            pltpu.SemaphoreType.DMA,
            pltpu.VMEM_SHARED((NPAD, FH), jnp.float32),
        ],
        mesh=mesh,
        compiler_params=pltpu.CompilerParams(use_tc_tiling_on_sc=False))


def _prep_body(x_ref, pos_ref, w1_ref, w2p_ref, b1_ref, b2_ref,
               u1_ref, wn1_ref, wn2_ref, cell_ref):
    xv = x_ref[...]
    pv = pos_ref[...]
    w1 = w1_ref[...]
    w2p = w2p_ref[...]
    px = pv[:, 0:1]
    py = pv[:, 1:2]
    p1 = px * w1[1:2, :] + py * w1[2:3, :]
    u1_ref[...] = xv * w1[0:1, :] + p1
    wn1_ref[...] = b1_ref[...] - p1
    wn2_ref[...] = b2_ref[...] - (px * w2p[0:1, :] + py * w2p[1:2, :])
    ix = jnp.clip(jnp.floor(px * CELL_INV), 0, GRID_W - 1).astype(jnp.int32)
    iy = jnp.clip(jnp.floor(py * CELL_INV), 0, GRID_W - 1).astype(jnp.int32)
    cell_ref[...] = iy * GRID_W + ix


def _mid_body(s_ref, deg_ref, wn1_ref, wn2_ref, w2h_ref, b2_ref,
              rm1_ref, rv1_ref, g1_ref, be1_ref, t2a_ref, t2b_ref):
    s = s_ref[...] + deg_ref[...] * wn1_ref[...]
    scale = g1_ref[...] * lax.rsqrt(rv1_ref[...] + EPS)
    h1 = jnp.maximum((s - rm1_ref[...]) * scale + be1_ref[...], 0.0)
    u2 = jnp.dot(h1, w2h_ref[...],
                 preferred_element_type=jnp.float32) + (b2_ref[...]
                                                        - wn2_ref[...])
    t2a_ref[...] = u2[:, :16]
    t2b_ref[...] = u2[:, 16:]


def _final_body(sa_ref, sb_ref, deg_ref, wn2_ref, cell_ref,
                rm2_ref, rv2_ref, g2_ref, be2_ref, outt_ref):
    pid = pl.program_id(0)
    agg = (jnp.concatenate([sa_ref[...], sb_ref[...]], axis=1)
           + deg_ref[...][:, 0:1] * wn2_ref[...])
    scale = g2_ref[...] * lax.rsqrt(rv2_ref[...] + EPS)
    h2 = jnp.maximum((agg - rm2_ref[...]) * scale + be2_ref[...], 0.0)

    rows = pid * BA + lax.broadcasted_iota(jnp.int32, (BA, 1), 0)
    valid = rows < N
    cids = lax.broadcasted_iota(jnp.int32, (1, NUM_GRIDS), 1)
    m = ((cell_ref[...] == cids) & valid).astype(jnp.float32)

    blockmax = jnp.stack(
        [jnp.max(m * h2[:, f:f + 1], axis=0) for f in range(32)], axis=0)

    @pl.when(pid == 0)
    def _():
        outt_ref[...] = blockmax

    @pl.when(pid > 0)
    def _():
        outt_ref[...] = jnp.maximum(outt_ref[...], blockmax)


def _row_spec(w):
    return pl.BlockSpec((BA, w), lambda i: (i, 0))


def _full_spec(shape):
    return pl.BlockSpec(shape, lambda i: tuple(0 for _ in shape))


def kernel(x, pos, edge_index, W1, b1, rm1, rv1, g1, beta1,
           W2, b2, rm2, rv2, g2, beta2):
    f32 = jnp.float32
    xpad = jnp.pad(x, ((0, NPAD - N), (0, 0)))
    pospad = jnp.pad(pos, ((0, NPAD - N), (0, 0)))
    srcg = jnp.concatenate(
        [edge_index[0], jnp.zeros((EPAD - E,), jnp.int32)]
    ).reshape(16, NBW, EBLK)
    dsts = jnp.concatenate(
        [edge_index[1], jnp.full((EPAD - E,), NPAD - 1, jnp.int32)]
    ).reshape(16, NBW, EBLK)
    z16 = jnp.zeros((RPT, FH), f32)
    onetab = jnp.ones((NPAD, FH), f32)

    seg = _make_seg_sum()
    (deg,) = seg(onetab, srcg, dsts, z16)

    u1, wn1, wn2, cell = pl.pallas_call(
        _prep_body,
        grid=(NBLK,),
        in_specs=[_row_spec(1), _row_spec(2), _full_spec((3, 16)),
                  _full_spec((2, 32)), _full_spec((1, 16)),
                  _full_spec((1, 32))],
        out_specs=[_row_spec(16), _row_spec(16), _row_spec(32), _row_spec(1)],
        out_shape=[jax.ShapeDtypeStruct((NPAD, 16), f32),
                   jax.ShapeDtypeStruct((NPAD, 16), f32),
                   jax.ShapeDtypeStruct((NPAD, 32), f32),
                   jax.ShapeDtypeStruct((NPAD, 1), jnp.int32)],
    )(xpad, pospad, W1, W2[16:18], b1.reshape(1, 16), b2.reshape(1, 32))

    (s1,) = seg(u1, srcg, dsts, z16)

    t2a, t2b = pl.pallas_call(
        _mid_body,
        grid=(NBLK,),
        in_specs=[_row_spec(16), _row_spec(16), _row_spec(16), _row_spec(32),
                  _full_spec((16, 32)), _full_spec((1, 32))]
                 + [_full_spec((1, 16))] * 4,
        out_specs=[_row_spec(16), _row_spec(16)],
        out_shape=[jax.ShapeDtypeStruct((NPAD, 16), f32),
                   jax.ShapeDtypeStruct((NPAD, 16), f32)],
    )(s1, deg, wn1, wn2, W2[:16], b2.reshape(1, 32),
      rm1.reshape(1, 16), rv1.reshape(1, 16),
      g1.reshape(1, 16), beta1.reshape(1, 16))

    (s2a,) = seg(t2a, srcg, dsts, z16)
    (s2b,) = seg(t2b, srcg, dsts, z16)

    outt = pl.pallas_call(
        _final_body,
        grid=(NBLK,),
        in_specs=[_row_spec(16), _row_spec(16), _row_spec(16), _row_spec(32),
                  _row_spec(1)] + [_full_spec((1, 32))] * 4,
        out_specs=_full_spec((32, NUM_GRIDS)),
        out_shape=jax.ShapeDtypeStruct((32, NUM_GRIDS), f32),
    )(s2a, s2b, deg, wn2, cell,
      rm2.reshape(1, 32), rv2.reshape(1, 32),
      g2.reshape(1, 32), beta2.reshape(1, 32))

    return outt.T


# speedup vs baseline: 1.0745x; 1.0745x over previous
import jax
import jax.numpy as jnp
from jax import lax
from jax.experimental import pallas as pl
from jax.experimental.pallas import tpu as pltpu
from jax.experimental.pallas import tpu_sc as plsc

N = 50000
E = 800000
CELL_INV = 1.0 / 16.0
GRID_W = 8
NUM_GRIDS = 64
EPS = 1e-5

NPAD = 51200
RPT = NPAD // 16
EPAD = 802816
EBLK = 512
NBW = EPAD // 16 // EBLK
NQ = 7
NBQ = NBW // NQ
BA = 2048
NBLK = NPAD // BA
FH = 16


def _make_seg_sum():
    mesh = plsc.VectorSubcoreMesh(core_axis_name="c", subcore_axis_name="s",
                                  num_cores=1)

    def body(tab_hbm, src_hbm, dst_hbm, z_hbm, s_out,
             srcv, dstv, rowsu, semu, sems, acc):
        t = lax.axis_index("s")
        pltpu.sync_copy(z_hbm, acc.at[pl.ds(t * RPT, RPT)])
        plsc.subcore_barrier()

        def chunk(q, carry0):
            pltpu.sync_copy(src_hbm.at[t, pl.ds(q * NBQ, NBQ)], srcv)
            pltpu.sync_copy(dst_hbm.at[t, pl.ds(q * NBQ, NBQ)], dstv)
            for p in range(3):
                pltpu.async_copy(tab_hbm.at[srcv.at[p]], rowsu.at[p], semu)

            def step(j, carry):
                @pl.when(j >= 1)
                def _():
                    pltpu.make_async_copy(
                        rowsu.at[0], acc.at[dstv.at[0]], sems).wait()

                pltpu.make_async_copy(
                    tab_hbm.at[srcv.at[j]], rowsu.at[j % 4], semu).wait()

                pltpu.async_copy(rowsu.at[j % 4], acc.at[dstv.at[j]],
                                 sems, add=True)

                @pl.when(j + 3 < NBQ)
                def _():
                    pltpu.async_copy(
                        tab_hbm.at[srcv.at[j + 3]], rowsu.at[(j + 3) % 4],
                        semu)

                return carry

            lax.fori_loop(0, NBQ, step, 0)
            pltpu.make_async_copy(rowsu.at[0], acc.at[dstv.at[0]],
                                  sems).wait()
            return carry0

        lax.fori_loop(0, NQ, chunk, 0)
        plsc.subcore_barrier()
        pltpu.sync_copy(acc.at[pl.ds(t * RPT, RPT)],
                        s_out.at[pl.ds(t * RPT, RPT)])

    return pl.kernel(
        body,
        out_type=(jax.ShapeDtypeStruct((NPAD, FH), jnp.float32),),
        scratch_types=[
            pltpu.VMEM((NBQ, EBLK), jnp.int32),
            pltpu.VMEM((NBQ, EBLK), jnp.int32),
            pltpu.VMEM((4, EBLK, FH), jnp.float32),
            pltpu.SemaphoreType.DMA,
            pltpu.SemaphoreType.DMA,
            pltpu.VMEM_SHARED((NPAD, FH), jnp.float32),
        ],
        mesh=mesh,
        compiler_params=pltpu.CompilerParams(use_tc_tiling_on_sc=False))


def _prep_body(x_ref, pos_ref, w1_ref, w2p_ref, b1_ref, b2_ref,
               u1_ref, wn1_ref, wn2_ref, cell_ref):
    xv = x_ref[...]
    pv = pos_ref[...]
    w1 = w1_ref[...]
    w2p = w2p_ref[...]
    px = pv[:, 0:1]
    py = pv[:, 1:2]
    p1 = px * w1[1:2, :] + py * w1[2:3, :]
    u1_ref[...] = xv * w1[0:1, :] + p1
    wn1_ref[...] = b1_ref[...] - p1
    wn2_ref[...] = b2_ref[...] - (px * w2p[0:1, :] + py * w2p[1:2, :])
    ix = jnp.clip(jnp.floor(px * CELL_INV), 0, GRID_W - 1).astype(jnp.int32)
    iy = jnp.clip(jnp.floor(py * CELL_INV), 0, GRID_W - 1).astype(jnp.int32)
    cell_ref[...] = iy * GRID_W + ix


def _mid_body(s_ref, deg_ref, wn1_ref, wn2_ref, w2h_ref, b2_ref,
              rm1_ref, rv1_ref, g1_ref, be1_ref, t2a_ref, t2b_ref):
    s = s_ref[...] + deg_ref[...] * wn1_ref[...]
    scale = g1_ref[...] * lax.rsqrt(rv1_ref[...] + EPS)
    h1 = jnp.maximum((s - rm1_ref[...]) * scale + be1_ref[...], 0.0)
    u2 = jnp.dot(h1, w2h_ref[...],
                 preferred_element_type=jnp.float32) + (b2_ref[...]
                                                        - wn2_ref[...])
    t2a_ref[...] = u2[:, :16]
    t2b_ref[...] = u2[:, 16:]


def _final_body(sa_ref, sb_ref, deg_ref, wn2_ref, cell_ref,
                rm2_ref, rv2_ref, g2_ref, be2_ref, outt_ref):
    pid = pl.program_id(0)
    agg = (jnp.concatenate([sa_ref[...], sb_ref[...]], axis=1)
           + deg_ref[...][:, 0:1] * wn2_ref[...])
    scale = g2_ref[...] * lax.rsqrt(rv2_ref[...] + EPS)
    h2 = jnp.maximum((agg - rm2_ref[...]) * scale + be2_ref[...], 0.0)

    rows = pid * BA + lax.broadcasted_iota(jnp.int32, (BA, 1), 0)
    valid = rows < N
    cids = lax.broadcasted_iota(jnp.int32, (1, NUM_GRIDS), 1)
    m = ((cell_ref[...] == cids) & valid).astype(jnp.float32)

    blockmax = jnp.stack(
        [jnp.max(m * h2[:, f:f + 1], axis=0) for f in range(32)], axis=0)

    @pl.when(pid == 0)
    def _():
        outt_ref[...] = blockmax

    @pl.when(pid > 0)
    def _():
        outt_ref[...] = jnp.maximum(outt_ref[...], blockmax)


def _row_spec(w):
    return pl.BlockSpec((BA, w), lambda i: (i, 0))


def _full_spec(shape):
    return pl.BlockSpec(shape, lambda i: tuple(0 for _ in shape))


def kernel(x, pos, edge_index, W1, b1, rm1, rv1, g1, beta1,
           W2, b2, rm2, rv2, g2, beta2):
    f32 = jnp.float32
    xpad = jnp.pad(x, ((0, NPAD - N), (0, 0)))
    pospad = jnp.pad(pos, ((0, NPAD - N), (0, 0)))
    srcg = jnp.concatenate(
        [edge_index[0], jnp.zeros((EPAD - E,), jnp.int32)]
    ).reshape(16, NBW, EBLK)
    dsts = jnp.concatenate(
        [edge_index[1], jnp.full((EPAD - E,), NPAD - 1, jnp.int32)]
    ).reshape(16, NBW, EBLK)
    z16 = jnp.zeros((RPT, FH), f32)
    onetab = jnp.ones((NPAD, FH), f32)

    seg = _make_seg_sum()
    (deg,) = seg(onetab, srcg, dsts, z16)

    u1, wn1, wn2, cell = pl.pallas_call(
        _prep_body,
        grid=(NBLK,),
        in_specs=[_row_spec(1), _row_spec(2), _full_spec((3, 16)),
                  _full_spec((2, 32)), _full_spec((1, 16)),
                  _full_spec((1, 32))],
        out_specs=[_row_spec(16), _row_spec(16), _row_spec(32), _row_spec(1)],
        out_shape=[jax.ShapeDtypeStruct((NPAD, 16), f32),
                   jax.ShapeDtypeStruct((NPAD, 16), f32),
                   jax.ShapeDtypeStruct((NPAD, 32), f32),
                   jax.ShapeDtypeStruct((NPAD, 1), jnp.int32)],
    )(xpad, pospad, W1, W2[16:18], b1.reshape(1, 16), b2.reshape(1, 32))

    (s1,) = seg(u1, srcg, dsts, z16)

    t2a, t2b = pl.pallas_call(
        _mid_body,
        grid=(NBLK,),
        in_specs=[_row_spec(16), _row_spec(16), _row_spec(16), _row_spec(32),
                  _full_spec((16, 32)), _full_spec((1, 32))]
                 + [_full_spec((1, 16))] * 4,
        out_specs=[_row_spec(16), _row_spec(16)],
        out_shape=[jax.ShapeDtypeStruct((NPAD, 16), f32),
                   jax.ShapeDtypeStruct((NPAD, 16), f32)],
    )(s1, deg, wn1, wn2, W2[:16], b2.reshape(1, 32),
      rm1.reshape(1, 16), rv1.reshape(1, 16),
      g1.reshape(1, 16), beta1.reshape(1, 16))

    (s2a,) = seg(t2a, srcg, dsts, z16)
    (s2b,) = seg(t2b, srcg, dsts, z16)

    outt = pl.pallas_call(
        _final_body,
        grid=(NBLK,),
        in_specs=[_row_spec(16), _row_spec(16), _row_spec(16), _row_spec(32),
                  _row_spec(1)] + [_full_spec((1, 32))] * 4,
        out_specs=_full_spec((32, NUM_GRIDS)),
        out_shape=jax.ShapeDtypeStruct((32, NUM_GRIDS), f32),
    )(s2a, s2b, deg, wn2, cell,
      rm2.reshape(1, 32), rv2.reshape(1, 32),
      g2.reshape(1, 32), beta2.reshape(1, 32))

    return outt.T


# speedup vs baseline: 1.0870x; 1.0116x over previous
import jax
import jax.numpy as jnp
from jax import lax
from jax.experimental import pallas as pl
from jax.experimental.pallas import tpu as pltpu
from jax.experimental.pallas import tpu_sc as plsc

N = 50000
E = 800000
CELL_INV = 1.0 / 16.0
GRID_W = 8
NUM_GRIDS = 64
EPS = 1e-5

NPAD = 51200
RPT = NPAD // 16
EPAD = 802816
EBLK = 512
NBW = EPAD // 16 // EBLK
NQ = 7
NBQ = NBW // NQ
BA = 2048
NBLK = NPAD // BA
FH = 16


def _make_seg_sum():
    mesh = plsc.VectorSubcoreMesh(core_axis_name="c", subcore_axis_name="s",
                                  num_cores=1)

    def body(tab_hbm, src_hbm, dst_hbm, z_hbm, s_out,
             srcv, dstv, rowsu, semu, sems, acc):
        t = lax.axis_index("s")
        pltpu.sync_copy(z_hbm, acc.at[pl.ds(t * RPT, RPT)])
        plsc.subcore_barrier()

        def chunk(q, carry0):
            pltpu.sync_copy(src_hbm.at[t, pl.ds(q * NBQ, NBQ)], srcv)
            pltpu.sync_copy(dst_hbm.at[t, pl.ds(q * NBQ, NBQ)], dstv)
            for p in range(5):
                pltpu.async_copy(tab_hbm.at[srcv.at[p]], rowsu.at[p], semu)

            def step(j, carry):
                @pl.when(j >= 1)
                def _():
                    pltpu.make_async_copy(
                        rowsu.at[0], acc.at[dstv.at[0]], sems).wait()

                pltpu.make_async_copy(
                    tab_hbm.at[srcv.at[j]], rowsu.at[j % 6], semu).wait()

                pltpu.async_copy(rowsu.at[j % 6], acc.at[dstv.at[j]],
                                 sems, add=True)

                @pl.when(j + 5 < NBQ)
                def _():
                    pltpu.async_copy(
                        tab_hbm.at[srcv.at[j + 5]], rowsu.at[(j + 5) % 6],
                        semu)

                return carry

            lax.fori_loop(0, NBQ, step, 0)
            pltpu.make_async_copy(rowsu.at[0], acc.at[dstv.at[0]],
                                  sems).wait()
            return carry0

        lax.fori_loop(0, NQ, chunk, 0)
        plsc.subcore_barrier()
        pltpu.sync_copy(acc.at[pl.ds(t * RPT, RPT)],
                        s_out.at[pl.ds(t * RPT, RPT)])

    return pl.kernel(
        body,
        out_type=(jax.ShapeDtypeStruct((NPAD, FH), jnp.float32),),
        scratch_types=[
            pltpu.VMEM((NBQ, EBLK), jnp.int32),
            pltpu.VMEM((NBQ, EBLK), jnp.int32),
            pltpu.VMEM((6, EBLK, FH), jnp.float32),
            pltpu.SemaphoreType.DMA,
            pltpu.SemaphoreType.DMA,
            pltpu.VMEM_SHARED((NPAD, FH), jnp.float32),
        ],
        mesh=mesh,
        compiler_params=pltpu.CompilerParams(use_tc_tiling_on_sc=False))


def _prep_body(x_ref, pos_ref, w1_ref, w2p_ref, b1_ref, b2_ref,
               u1_ref, wn1_ref, wn2_ref, cell_ref):
    xv = x_ref[...]
    pv = pos_ref[...]
    w1 = w1_ref[...]
    w2p = w2p_ref[...]
    px = pv[:, 0:1]
    py = pv[:, 1:2]
    p1 = px * w1[1:2, :] + py * w1[2:3, :]
    u1_ref[...] = xv * w1[0:1, :] + p1
    wn1_ref[...] = b1_ref[...] - p1
    wn2_ref[...] = b2_ref[...] - (px * w2p[0:1, :] + py * w2p[1:2, :])
    ix = jnp.clip(jnp.floor(px * CELL_INV), 0, GRID_W - 1).astype(jnp.int32)
    iy = jnp.clip(jnp.floor(py * CELL_INV), 0, GRID_W - 1).astype(jnp.int32)
    cell_ref[...] = iy * GRID_W + ix


def _mid_body(s_ref, deg_ref, wn1_ref, wn2_ref, w2h_ref, b2_ref,
              rm1_ref, rv1_ref, g1_ref, be1_ref, t2a_ref, t2b_ref):
    s = s_ref[...] + deg_ref[...] * wn1_ref[...]
    scale = g1_ref[...] * lax.rsqrt(rv1_ref[...] + EPS)
    h1 = jnp.maximum((s - rm1_ref[...]) * scale + be1_ref[...], 0.0)
    u2 = jnp.dot(h1, w2h_ref[...],
                 preferred_element_type=jnp.float32) + (b2_ref[...]
                                                        - wn2_ref[...])
    t2a_ref[...] = u2[:, :16]
    t2b_ref[...] = u2[:, 16:]


def _final_body(sa_ref, sb_ref, deg_ref, wn2_ref, cell_ref,
                rm2_ref, rv2_ref, g2_ref, be2_ref, outt_ref):
    pid = pl.program_id(0)
    agg = (jnp.concatenate([sa_ref[...], sb_ref[...]], axis=1)
           + deg_ref[...][:, 0:1] * wn2_ref[...])
    scale = g2_ref[...] * lax.rsqrt(rv2_ref[...] + EPS)
    h2 = jnp.maximum((agg - rm2_ref[...]) * scale + be2_ref[...], 0.0)

    rows = pid * BA + lax.broadcasted_iota(jnp.int32, (BA, 1), 0)
    valid = rows < N
    cids = lax.broadcasted_iota(jnp.int32, (1, NUM_GRIDS), 1)
    m = ((cell_ref[...] == cids) & valid).astype(jnp.float32)

    blockmax = jnp.stack(
        [jnp.max(m * h2[:, f:f + 1], axis=0) for f in range(32)], axis=0)

    @pl.when(pid == 0)
    def _():
        outt_ref[...] = blockmax

    @pl.when(pid > 0)
    def _():
        outt_ref[...] = jnp.maximum(outt_ref[...], blockmax)


def _row_spec(w):
    return pl.BlockSpec((BA, w), lambda i: (i, 0))


def _full_spec(shape):
    return pl.BlockSpec(shape, lambda i: tuple(0 for _ in shape))


def kernel(x, pos, edge_index, W1, b1, rm1, rv1, g1, beta1,
           W2, b2, rm2, rv2, g2, beta2):
    f32 = jnp.float32
    xpad = jnp.pad(x, ((0, NPAD - N), (0, 0)))
    pospad = jnp.pad(pos, ((0, NPAD - N), (0, 0)))
    srcg = jnp.concatenate(
        [edge_index[0], jnp.zeros((EPAD - E,), jnp.int32)]
    ).reshape(16, NBW, EBLK)
    dsts = jnp.concatenate(
        [edge_index[1], jnp.full((EPAD - E,), NPAD - 1, jnp.int32)]
    ).reshape(16, NBW, EBLK)
    z16 = jnp.zeros((RPT, FH), f32)
    onetab = jnp.ones((NPAD, FH), f32)

    seg = _make_seg_sum()
    (deg,) = seg(onetab, srcg, dsts, z16)

    u1, wn1, wn2, cell = pl.pallas_call(
        _prep_body,
        grid=(NBLK,),
        in_specs=[_row_spec(1), _row_spec(2), _full_spec((3, 16)),
                  _full_spec((2, 32)), _full_spec((1, 16)),
                  _full_spec((1, 32))],
        out_specs=[_row_spec(16), _row_spec(16), _row_spec(32), _row_spec(1)],
        out_shape=[jax.ShapeDtypeStruct((NPAD, 16), f32),
                   jax.ShapeDtypeStruct((NPAD, 16), f32),
                   jax.ShapeDtypeStruct((NPAD, 32), f32),
                   jax.ShapeDtypeStruct((NPAD, 1), jnp.int32)],
    )(xpad, pospad, W1, W2[16:18], b1.reshape(1, 16), b2.reshape(1, 32))

    (s1,) = seg(u1, srcg, dsts, z16)

    t2a, t2b = pl.pallas_call(
        _mid_body,
        grid=(NBLK,),
        in_specs=[_row_spec(16), _row_spec(16), _row_spec(16), _row_spec(32),
                  _full_spec((16, 32)), _full_spec((1, 32))]
                 + [_full_spec((1, 16))] * 4,
        out_specs=[_row_spec(16), _row_spec(16)],
        out_shape=[jax.ShapeDtypeStruct((NPAD, 16), f32),
                   jax.ShapeDtypeStruct((NPAD, 16), f32)],
    )(s1, deg, wn1, wn2, W2[:16], b2.reshape(1, 32),
      rm1.reshape(1, 16), rv1.reshape(1, 16),
      g1.reshape(1, 16), beta1.reshape(1, 16))

    (s2a,) = seg(t2a, srcg, dsts, z16)
    (s2b,) = seg(t2b, srcg, dsts, z16)

    outt = pl.pallas_call(
        _final_body,
        grid=(NBLK,),
        in_specs=[_row_spec(16), _row_spec(16), _row_spec(16), _row_spec(32),
                  _row_spec(1)] + [_full_spec((1, 32))] * 4,
        out_specs=_full_spec((32, NUM_GRIDS)),
        out_shape=jax.ShapeDtypeStruct((32, NUM_GRIDS), f32),
    )(s2a, s2b, deg, wn2, cell,
      rm2.reshape(1, 32), rv2.reshape(1, 32),
      g2.reshape(1, 32), beta2.reshape(1, 32))

    return outt.T
